# Initial kernel scaffold; baseline (speedup 1.0000x reference)
import jax
import jax.numpy as jnp
from jax.experimental import pallas as pl


def kernel(x, edge_index, batch, Wl1, bl1, Wr1, Wl2, bl2, Wr2, W_lin1, b_lin1, W_lin2, b_lin2):
    raise NotImplementedError("write your pallas kernel here")


# speedup vs baseline: 5.1684x; 5.1684x over previous
import functools

import jax
import jax.numpy as jnp
from jax import lax
from jax.experimental import pallas as pl
from jax.experimental.pallas import tpu as pltpu
from jax.experimental.pallas import tpu_sc as plsc

N = 10000
E = 320000
D = 128
G = 64

NC = 2
NS = 16
NW = NC * NS
EPW = E // NW
CH = 80
NCHUNK = EPW // CH
DEGW = 16
RB = 80
NRB = N // RB


def _sc_agg_body(x_hbm, src_hbm, dst_hbm, agg_out, rows_v, src_v, dst_v,
                 acc_sh, gsem):
    c = lax.axis_index("c")
    s = lax.axis_index("s")
    wid = c * NS + s

    def zrow(t, _):
        rows_v[t // 8, pl.ds((t % 8) * 16, 16)] = jnp.zeros((16,), jnp.float32)
        return 0
    lax.fori_loop(0, CH * 8, zrow, 0)

    for j in range(8):
        k = s + 16 * j
        @pl.when(k < NRB)
        def _():
            pltpu.sync_copy(rows_v, acc_sh.at[pl.ds(k * RB, RB)])
    plsc.subcore_barrier()

    def step(i, _):
        base = wid * EPW + i * CH
        pltpu.sync_copy(src_hbm.at[pl.ds(base, CH)], src_v)
        pltpu.async_copy(x_hbm.at[src_v], rows_v, gsem).wait()
        pltpu.sync_copy(dst_hbm.at[pl.ds(base, CH)], dst_v)
        pltpu.sync_copy(rows_v, acc_sh.at[dst_v], add=True)
        return 0
    lax.fori_loop(0, NCHUNK, step, 0)
    plsc.subcore_barrier()

    for j in range(8):
        k = s + 16 * j
        @pl.when(k < NRB)
        def _():
            pltpu.sync_copy(acc_sh.at[pl.ds(k * RB, RB)], rows_v)
            pltpu.sync_copy(rows_v, agg_out.at[pl.ds(c * N + k * RB, RB)])


_sc_agg = pl.kernel(
    _sc_agg_body,
    out_type=jax.ShapeDtypeStruct((NC * N, D), jnp.float32),
    mesh=plsc.VectorSubcoreMesh(core_axis_name="c", subcore_axis_name="s"),
    scratch_types=[
        pltpu.VMEM((CH, D), jnp.float32),
        pltpu.VMEM((CH,), jnp.int32),
        pltpu.VMEM((CH,), jnp.int32),
        pltpu.VMEM_SHARED((N, D), jnp.float32),
        pltpu.SemaphoreType.DMA,
    ],
)

NH = 80
_EB = 4000


def _deg_body(dst_ref, o_ref, acc_ref):
    i = pl.program_id(0)

    @pl.when(i == 0)
    def _():
        acc_ref[...] = jnp.zeros_like(acc_ref)

    dcol = dst_ref[0]
    hi = lax.div(dcol, 128)
    lo = dcol - hi * 128
    oh_hi = (hi == lax.broadcasted_iota(jnp.int32, (1, NH), 1)
             ).astype(jnp.float32)
    oh_lo = (lo == lax.broadcasted_iota(jnp.int32, (1, 128), 1)
             ).astype(jnp.float32)
    acc_ref[...] += lax.dot_general(oh_hi, oh_lo, (((0,), (0,)), ((), ())),
                                    preferred_element_type=jnp.float32)

    @pl.when(i == pl.num_programs(0) - 1)
    def _():
        o_ref[...] = acc_ref[...]


def _tc_deg(dst3):
    return pl.pallas_call(
        _deg_body,
        grid=(E // _EB,),
        in_specs=[pl.BlockSpec((1, _EB, 1), lambda i: (i, 0, 0))],
        out_specs=pl.BlockSpec((NH, 128), lambda i: (0, 0)),
        out_shape=jax.ShapeDtypeStruct((NH, 128), jnp.float32),
        scratch_shapes=[pltpu.VMEM((NH, 128), jnp.float32)],
    )(dst3)


def _layer_body(a0_ref, a1_ref, d_ref, x_ref, wl_ref, bl_ref,
                wr_ref, o_ref):
    a = a0_ref[...] + a1_ref[...]
    d = d_ref[0]
    mean = a / jnp.maximum(d, 1.0)
    z = lax.dot_general(mean, wl_ref[...], (((1,), (1,)), ((), ())),
                        preferred_element_type=jnp.float32)
    z = z + lax.dot_general(x_ref[...], wr_ref[...], (((1,), (1,)), ((), ())),
                            preferred_element_type=jnp.float32)
    z = z + bl_ref[...]
    o_ref[...] = jnp.where(z > 0, z, jnp.exp(jnp.minimum(z, 0.0)) - 1.0)


_LB = 1000


def _tc_layer(agg, deg, x, wl, bl, wr):
    grid = (N // _LB,)
    return pl.pallas_call(
        _layer_body,
        grid=grid,
        in_specs=[
            pl.BlockSpec((_LB, D), lambda i: (i, 0)),
            pl.BlockSpec((_LB, D), lambda i: (N // _LB + i, 0)),
            pl.BlockSpec((1, _LB, 1), lambda i: (i, 0, 0)),
            pl.BlockSpec((_LB, D), lambda i: (i, 0)),
            pl.BlockSpec((D, D), lambda i: (0, 0)),
            pl.BlockSpec((1, D), lambda i: (0, 0)),
            pl.BlockSpec((D, D), lambda i: (0, 0)),
        ],
        out_specs=pl.BlockSpec((_LB, D), lambda i: (i, 0)),
        out_shape=jax.ShapeDtypeStruct((N, D), jnp.float32),
    )(agg, agg, deg, x, wl, bl, wr)


def _pool_body(x_ref, bv_ref, bs_ref, w1_ref, b1_ref, w2_ref, b2_ref, o_ref,
               sum_ref, max_ref, cnt_ref):
    i = pl.program_id(0)
    nb = pl.num_programs(0)

    @pl.when(i == 0)
    def _():
        sum_ref[...] = jnp.zeros_like(sum_ref)
        cnt_ref[...] = jnp.zeros_like(cnt_ref)
        max_ref[...] = jnp.full_like(max_ref, -jnp.inf)

    xb = x_ref[...]
    bcol = bv_ref[0]
    gio = lax.broadcasted_iota(jnp.int32, (G, 1), 0)
    grow = lax.broadcasted_iota(jnp.int32, (1, G), 1)
    onehot = (bcol == grow).astype(jnp.float32)
    sum_ref[...] += lax.dot_general(onehot, xb, (((0,), (0,)), ((), ())),
                                    preferred_element_type=jnp.float32)
    cnt_ref[...] += lax.dot_general(onehot, jnp.ones((_LB, 1), jnp.float32),
                                    (((0,), (0,)), ((), ())),
                                    preferred_element_type=jnp.float32)

    bmin = bs_ref[0, 0, 0]
    bmax = bs_ref[0, 0, _LB - 1]

    def upd(g, _):
        m = (bcol == g)
        colv = jnp.max(jnp.where(m, xb, -jnp.inf),
                       axis=0, keepdims=True)
        rm = (gio == g)
        max_ref[...] = jnp.where(rm, jnp.maximum(max_ref[...], colv),
                                 max_ref[...])
        return 0
    lax.fori_loop(bmin, bmax + 1, upd, 0)

    @pl.when(i == nb - 1)
    def _():
        pooled = sum_ref[...] / jnp.maximum(cnt_ref[...], 1.0) + max_ref[...]
        h = lax.dot_general(pooled, w1_ref[...], (((1,), (1,)), ((), ())),
                            preferred_element_type=jnp.float32) + b1_ref[...]
        h = jnp.maximum(h, 0.0)
        o_ref[...] = lax.dot_general(h, w2_ref[...], (((1,), (1,)), ((), ())),
                                     preferred_element_type=jnp.float32) + b2_ref[...]


def _tc_pool_mlp(x2, batchc, batch3, w1, b1, w2, b2):
    grid = (N // _LB,)
    return pl.pallas_call(
        _pool_body,
        grid=grid,
        in_specs=[
            pl.BlockSpec((_LB, D), lambda i: (i, 0)),
            pl.BlockSpec((1, _LB, 1), lambda i: (i, 0, 0)),
            pl.BlockSpec((1, 1, _LB), lambda i: (i, 0, 0),
                         memory_space=pltpu.SMEM),
            pl.BlockSpec((D, D), lambda i: (0, 0)),
            pl.BlockSpec((1, D), lambda i: (0, 0)),
            pl.BlockSpec((D, D), lambda i: (0, 0)),
            pl.BlockSpec((1, D), lambda i: (0, 0)),
        ],
        out_specs=pl.BlockSpec((G, D), lambda i: (0, 0)),
        out_shape=jax.ShapeDtypeStruct((G, D), jnp.float32),
        scratch_shapes=[
            pltpu.VMEM((G, D), jnp.float32),
            pltpu.VMEM((G, D), jnp.float32),
            pltpu.VMEM((G, 1), jnp.float32),
        ],
    )(x2, batchc, batch3, w1, b1, w2, b2)


def kernel(x, edge_index, batch, Wl1, bl1, Wr1, Wl2, bl2, Wr2,
           W_lin1, b_lin1, W_lin2, b_lin2):
    src = edge_index[0].astype(jnp.int32)
    dst = edge_index[1].astype(jnp.int32)
    deg = _tc_deg(dst.reshape(E // _EB, _EB, 1))
    deg = deg.reshape(NH * 128)[:N].reshape(N // _LB, _LB, 1)
    agg1 = _sc_agg(x, src, dst)
    x1 = _tc_layer(agg1, deg, x, Wl1, bl1.reshape(1, D), Wr1)
    agg2 = _sc_agg(x1, src, dst)
    x2 = _tc_layer(agg2, deg, x1, Wl2, bl2.reshape(1, D), Wr2)
    b32 = batch.astype(jnp.int32)
    batchc = b32.reshape(N // _LB, _LB, 1)
    batch3 = b32.reshape(N // _LB, 1, _LB)
    return _tc_pool_mlp(x2, batchc, batch3, W_lin1, b_lin1.reshape(1, D),
                        W_lin2, b_lin2.reshape(1, D))
